# Initial kernel scaffold; baseline (speedup 1.0000x reference)
import jax
import jax.numpy as jnp
from jax.experimental import pallas as pl


def kernel(inputs, targets, superpixels, spmasks):
    raise NotImplementedError("write your pallas kernel here")


# speedup vs baseline: 13.4145x; 13.4145x over previous
import functools

import jax
import jax.numpy as jnp
from jax import lax
from jax.experimental import pallas as pl
from jax.experimental.pallas import tpu as pltpu
from jax.experimental.pallas import tpu_sc as plsc

EPS = 1e-08

N_IMG = 4
C = 20
HW = 512 * 512
NSP = 2048

_NC = 2
_NS = 16
_NW = _NC * _NS
_CHUNK = (N_IMG * HW) // _NW
_W_PER_IMG = HW // _CHUNK
_LANES = 16

_B = 8192
_NBLK = HW // _B


def _pack_body(t_ref, bits_ref):
    t = t_ref[...]
    cidx = lax.broadcasted_iota(jnp.int32, t.shape, 2)
    tbit = (t != 0.0).astype(jnp.int32)
    bits_ref[...] = jnp.sum(tbit << cidx, axis=2)


def _pack_targets(targets):
    return pl.pallas_call(
        _pack_body,
        out_shape=jax.ShapeDtypeStruct((N_IMG, NSP), jnp.int32),
    )(targets)


@functools.cache
def _make_sc_gather():
    mesh = plsc.VectorSubcoreMesh(core_axis_name="c", subcore_axis_name="s")

    @functools.partial(
        pl.kernel,
        mesh=mesh,
        out_type=jax.ShapeDtypeStruct((N_IMG * HW,), jnp.int32),
        scratch_types=[
            pltpu.VMEM((NSP,), jnp.int32),
            pltpu.VMEM((_CHUNK,), jnp.int32),
            pltpu.VMEM((_CHUNK,), jnp.int32),
            pltpu.VMEM((_CHUNK,), jnp.int32),
        ],
        compiler_params=pltpu.CompilerParams(needs_layout_passes=False),
    )
    def sc_gather(bits_hbm, sp_hbm, spm_hbm, out_hbm, table_v, idx_v, spm_v, out_v):
        wid = lax.axis_index("s") * _NC + lax.axis_index("c")
        img = wid // _W_PER_IMG
        base = pl.multiple_of(wid * _CHUNK, 8)
        tab_off = pl.multiple_of(img * NSP, 8)
        pltpu.sync_copy(bits_hbm.at[pl.ds(tab_off, NSP)], table_v)
        pltpu.sync_copy(sp_hbm.at[pl.ds(base, _CHUNK)], idx_v)
        pltpu.sync_copy(spm_hbm.at[pl.ds(base, _CHUNK)], spm_v)

        def body(i, carry):
            b = i * _LANES
            idx = idx_v[pl.ds(b, _LANES)]
            g = plsc.load_gather(table_v, [idx])
            out_v[pl.ds(b, _LANES)] = g * spm_v[pl.ds(b, _LANES)]
            return carry

        lax.fori_loop(0, _CHUNK // _LANES, body, 0)
        pltpu.sync_copy(out_v, out_hbm.at[pl.ds(base, _CHUNK)])

    return sc_gather


def _math_body(x_ref, m_ref, out_ref, acc_ref, cnt_ref):
    step = pl.program_id(0) * _NBLK + pl.program_id(1)

    @pl.when(step == 0)
    def _():
        acc_ref[0] = jnp.float32(0.0)
        cnt_ref[0] = jnp.int32(0)

    x = x_ref[0]
    m = m_ref[0]
    mb = jnp.broadcast_to(m, (C, _B))
    cbit = lax.broadcasted_iota(jnp.int32, (C, _B), 0)
    t = ((mb >> cbit) & 1).astype(jnp.float32)

    e = jnp.exp(x)
    s0 = jnp.sum(e * (1.0 - t), axis=0, keepdims=True)
    denom = (s0 + e) * t
    exsm = e * t / (denom + EPS)
    ce = -jnp.log(exsm + EPS) * t
    pix = jnp.sum(ce, axis=0, keepdims=True)
    k = jnp.sum(t, axis=0, keepdims=True)
    sel = m != 0
    pix_ce = jnp.where(sel, pix / jnp.maximum(k, 1.0), 0.0)

    acc_ref[0] += jnp.sum(pix_ce)
    cnt_ref[0] += jnp.sum(sel.astype(jnp.int32))

    @pl.when(step == N_IMG * _NBLK - 1)
    def _():
        out_ref[0, 0] = acc_ref[0] / (jnp.int32(1) + cnt_ref[0]).astype(jnp.float32)


def _math(x3, m3):
    return pl.pallas_call(
        _math_body,
        grid=(N_IMG, _NBLK),
        in_specs=[
            pl.BlockSpec((1, C, _B), lambda n, b: (n, 0, b)),
            pl.BlockSpec((1, 1, _B), lambda n, b: (n * _NBLK + b, 0, 0)),
        ],
        out_specs=pl.BlockSpec((1, 1), lambda n, b: (0, 0), memory_space=pltpu.SMEM),
        out_shape=jax.ShapeDtypeStruct((1, 1), jnp.float32),
        scratch_shapes=[
            pltpu.SMEM((1,), jnp.float32),
            pltpu.SMEM((1,), jnp.int32),
        ],
    )(x3, m3)


def kernel(inputs, targets, superpixels, spmasks):
    n, c, h, w = inputs.shape
    x3 = inputs.reshape(n, c, h * w)
    sp = superpixels.reshape(-1).astype(jnp.int32)
    spm = spmasks.reshape(-1).astype(jnp.int32)

    bits = _pack_targets(targets).reshape(-1)
    masked = _make_sc_gather()(bits, sp, spm)
    m3 = masked.reshape(n * _NBLK, 1, _B)
    out = _math(x3, m3)
    return out[0, 0]


# speedup vs baseline: 14.4235x; 1.0752x over previous
import functools

import jax
import jax.numpy as jnp
from jax import lax
from jax.experimental import pallas as pl
from jax.experimental.pallas import tpu as pltpu
from jax.experimental.pallas import tpu_sc as plsc

EPS = 1e-08

N_IMG = 4
C = 20
HW = 512 * 512
NSP = 2048

_NC = 2
_NS = 16
_NW = _NC * _NS
_CHUNK = (N_IMG * HW) // _NW
_W_PER_IMG = HW // _CHUNK
_LANES = 16

_B = 8192
_NBLK = HW // _B


def _pack_body(t_ref, bits_ref):
    t = t_ref[...]
    cidx = lax.broadcasted_iota(jnp.int32, t.shape, 2)
    tbit = (t != 0.0).astype(jnp.int32)
    bits_ref[...] = jnp.sum(tbit << cidx, axis=2)


def _pack_targets(targets):
    return pl.pallas_call(
        _pack_body,
        out_shape=jax.ShapeDtypeStruct((N_IMG, NSP), jnp.int32),
    )(targets)


@functools.cache
def _make_sc_gather():
    mesh = plsc.VectorSubcoreMesh(core_axis_name="c", subcore_axis_name="s")

    @functools.partial(
        pl.kernel,
        mesh=mesh,
        out_type=jax.ShapeDtypeStruct((N_IMG * HW,), jnp.int32),
        scratch_types=[
            pltpu.VMEM((NSP,), jnp.int32),
            pltpu.VMEM((_CHUNK,), jnp.int32),
            pltpu.VMEM((_CHUNK,), jnp.int32),
            pltpu.VMEM((_CHUNK,), jnp.int32),
        ],
        compiler_params=pltpu.CompilerParams(needs_layout_passes=False),
    )
    def sc_gather(bits_hbm, sp_hbm, spm_hbm, out_hbm, table_v, idx_v, spm_v, out_v):
        wid = lax.axis_index("s") * _NC + lax.axis_index("c")
        img = wid // _W_PER_IMG
        base = pl.multiple_of(wid * _CHUNK, 8)
        tab_off = pl.multiple_of(img * NSP, 8)
        pltpu.sync_copy(bits_hbm.at[pl.ds(tab_off, NSP)], table_v)
        pltpu.sync_copy(sp_hbm.at[pl.ds(base, _CHUNK)], idx_v)
        pltpu.sync_copy(spm_hbm.at[pl.ds(base, _CHUNK)], spm_v)

        def body(i, carry):
            b = i * _LANES
            idx = idx_v[pl.ds(b, _LANES)]
            g = plsc.load_gather(table_v, [idx])
            out_v[pl.ds(b, _LANES)] = g * spm_v[pl.ds(b, _LANES)]
            return carry

        lax.fori_loop(0, _CHUNK // _LANES, body, 0)
        pltpu.sync_copy(out_v, out_hbm.at[pl.ds(base, _CHUNK)])

    return sc_gather


def _math_body(x_ref, m_ref, out_ref, acc_ref, cnt_ref):
    step = pl.program_id(0) * _NBLK + pl.program_id(1)

    @pl.when(step == 0)
    def _():
        acc_ref[0] = jnp.float32(0.0)
        cnt_ref[0] = jnp.int32(0)

    x = x_ref[0]
    m = m_ref[0]
    mb = jnp.broadcast_to(m, (C, _B))
    cbit = lax.broadcasted_iota(jnp.int32, (C, _B), 0)
    t = ((mb >> cbit) & 1).astype(jnp.float32)

    e = jnp.exp(x)
    s0 = jnp.sum(e * (1.0 - t), axis=0, keepdims=True)
    ce = (jnp.log(s0 + e) - x) * t
    pix = jnp.sum(ce, axis=0, keepdims=True)
    k = jnp.sum(t, axis=0, keepdims=True)
    sel = m != 0
    pix_ce = jnp.where(sel, pix / jnp.maximum(k, 1.0), 0.0)

    acc_ref[0] += jnp.sum(pix_ce)
    cnt_ref[0] += jnp.sum(sel.astype(jnp.int32))

    @pl.when(step == N_IMG * _NBLK - 1)
    def _():
        out_ref[0, 0] = acc_ref[0] / (jnp.int32(1) + cnt_ref[0]).astype(jnp.float32)


def _math(x3, m3):
    return pl.pallas_call(
        _math_body,
        grid=(N_IMG, _NBLK),
        in_specs=[
            pl.BlockSpec((1, C, _B), lambda n, b: (n, 0, b)),
            pl.BlockSpec((1, 1, _B), lambda n, b: (n * _NBLK + b, 0, 0)),
        ],
        out_specs=pl.BlockSpec((1, 1), lambda n, b: (0, 0), memory_space=pltpu.SMEM),
        out_shape=jax.ShapeDtypeStruct((1, 1), jnp.float32),
        scratch_shapes=[
            pltpu.SMEM((1,), jnp.float32),
            pltpu.SMEM((1,), jnp.int32),
        ],
    )(x3, m3)


def kernel(inputs, targets, superpixels, spmasks):
    n, c, h, w = inputs.shape
    x3 = inputs.reshape(n, c, h * w)
    sp = superpixels.reshape(-1).astype(jnp.int32)
    spm = spmasks.reshape(-1).astype(jnp.int32)

    bits = _pack_targets(targets).reshape(-1)
    masked = _make_sc_gather()(bits, sp, spm)
    m3 = masked.reshape(n * _NBLK, 1, _B)
    out = _math(x3, m3)
    return out[0, 0]


# speedup vs baseline: 15.0163x; 1.0411x over previous
import functools

import jax
import jax.numpy as jnp
from jax import lax
from jax.experimental import pallas as pl
from jax.experimental.pallas import tpu as pltpu
from jax.experimental.pallas import tpu_sc as plsc

EPS = 1e-08

N_IMG = 4
C = 20
HW = 512 * 512
NSP = 2048

_NC = 2
_NS = 16
_NW = _NC * _NS
_CHUNK = (N_IMG * HW) // _NW
_W_PER_IMG = HW // _CHUNK
_LANES = 16

_B = 32768
_NBLK = HW // _B


def _pack_body(t_ref, bits_ref):
    t = t_ref[...].reshape(N_IMG * NSP, C)
    w = jnp.exp2(lax.broadcasted_iota(jnp.int32, (C, 1), 0).astype(jnp.float32))
    tbit = (t != 0.0).astype(jnp.float32)
    bits = jax.lax.dot_general(tbit, w, (((1,), (0,)), ((), ())),
                               preferred_element_type=jnp.float32)
    bits_ref[...] = bits.astype(jnp.int32).reshape(N_IMG, NSP)


def _pack_targets(targets):
    return pl.pallas_call(
        _pack_body,
        out_shape=jax.ShapeDtypeStruct((N_IMG, NSP), jnp.int32),
    )(targets)


@functools.cache
def _make_sc_gather():
    mesh = plsc.VectorSubcoreMesh(core_axis_name="c", subcore_axis_name="s")

    @functools.partial(
        pl.kernel,
        mesh=mesh,
        out_type=jax.ShapeDtypeStruct((N_IMG * HW,), jnp.int32),
        scratch_types=[
            pltpu.VMEM((NSP,), jnp.int32),
            pltpu.VMEM((_CHUNK,), jnp.int32),
            pltpu.VMEM((_CHUNK,), jnp.int32),
            pltpu.VMEM((_CHUNK,), jnp.int32),
        ],
        compiler_params=pltpu.CompilerParams(needs_layout_passes=False),
    )
    def sc_gather(bits_hbm, sp_hbm, spm_hbm, out_hbm, table_v, idx_v, spm_v, out_v):
        wid = lax.axis_index("s") * _NC + lax.axis_index("c")
        img = wid // _W_PER_IMG
        base = pl.multiple_of(wid * _CHUNK, 8)
        tab_off = pl.multiple_of(img * NSP, 8)
        pltpu.sync_copy(bits_hbm.at[pl.ds(tab_off, NSP)], table_v)
        pltpu.sync_copy(sp_hbm.at[pl.ds(base, _CHUNK)], idx_v)
        pltpu.sync_copy(spm_hbm.at[pl.ds(base, _CHUNK)], spm_v)

        def body(i, carry):
            b = i * _LANES
            idx = idx_v[pl.ds(b, _LANES)]
            g = plsc.load_gather(table_v, [idx])
            out_v[pl.ds(b, _LANES)] = g * spm_v[pl.ds(b, _LANES)]
            return carry

        lax.fori_loop(0, _CHUNK // _LANES, body, 0)
        pltpu.sync_copy(out_v, out_hbm.at[pl.ds(base, _CHUNK)])

    return sc_gather


def _math_body(x_ref, m_ref, out_ref, acc_ref, cnt_ref):
    step = pl.program_id(0) * _NBLK + pl.program_id(1)

    @pl.when(step == 0)
    def _():
        acc_ref[0] = jnp.float32(0.0)
        cnt_ref[0] = jnp.int32(0)

    x = x_ref[0]
    m = m_ref[0]
    mb = jnp.broadcast_to(m, (C, _B))
    cbit = lax.broadcasted_iota(jnp.int32, (C, _B), 0)
    t = ((mb >> cbit) & 1).astype(jnp.float32)

    e = jnp.exp(x)
    s0 = jnp.sum(e * (1.0 - t), axis=0, keepdims=True)
    ce = (jnp.log(s0 + e) - x) * t
    pix = jnp.sum(ce, axis=0, keepdims=True)
    k = jnp.sum(t, axis=0, keepdims=True)
    sel = m != 0
    pix_ce = jnp.where(sel, pix / jnp.maximum(k, 1.0), 0.0)

    acc_ref[0] += jnp.sum(pix_ce)
    cnt_ref[0] += jnp.sum(sel.astype(jnp.int32))

    @pl.when(step == N_IMG * _NBLK - 1)
    def _():
        out_ref[0, 0] = acc_ref[0] / (jnp.int32(1) + cnt_ref[0]).astype(jnp.float32)


def _math(x3, m3):
    return pl.pallas_call(
        _math_body,
        grid=(N_IMG, _NBLK),
        in_specs=[
            pl.BlockSpec((1, C, _B), lambda n, b: (n, 0, b)),
            pl.BlockSpec((1, 1, _B), lambda n, b: (n * _NBLK + b, 0, 0)),
        ],
        out_specs=pl.BlockSpec((1, 1), lambda n, b: (0, 0), memory_space=pltpu.SMEM),
        out_shape=jax.ShapeDtypeStruct((1, 1), jnp.float32),
        scratch_shapes=[
            pltpu.SMEM((1,), jnp.float32),
            pltpu.SMEM((1,), jnp.int32),
        ],
    )(x3, m3)


def kernel(inputs, targets, superpixels, spmasks):
    n, c, h, w = inputs.shape
    x3 = inputs.reshape(n, c, h * w)
    sp = superpixels.reshape(-1).astype(jnp.int32)
    spm = spmasks.reshape(-1).astype(jnp.int32)

    bits = _pack_targets(targets).reshape(-1)
    masked = _make_sc_gather()(bits, sp, spm)
    m3 = masked.reshape(n * _NBLK, 1, _B)
    out = _math(x3, m3)
    return out[0, 0]


# speedup vs baseline: 30.3141x; 2.0187x over previous
import functools

import jax
import jax.numpy as jnp
from jax import lax
from jax.experimental import pallas as pl
from jax.experimental.pallas import tpu as pltpu
from jax.experimental.pallas import tpu_sc as plsc

EPS = 1e-08

N_IMG = 4
C = 20
HW = 512 * 512
NSP = 2048

_NC = 2
_NS = 16
_NW = _NC * _NS
_CHUNK = (N_IMG * HW) // _NW
_W_PER_IMG = HW // _CHUNK
_LANES = 16

_S = 64
_B = _S * 512
_NBLK = HW // _B


def _pack_body(t_ref, bits_ref):
    t = t_ref[...].reshape(N_IMG * NSP, C)
    w = jnp.exp2(lax.broadcasted_iota(jnp.int32, (C, 1), 0).astype(jnp.float32))
    tbit = (t != 0.0).astype(jnp.float32)
    bits = jax.lax.dot_general(tbit, w, (((1,), (0,)), ((), ())),
                               preferred_element_type=jnp.float32)
    bits_ref[...] = bits.astype(jnp.int32).reshape(N_IMG, NSP)


def _pack_targets(targets):
    return pl.pallas_call(
        _pack_body,
        out_shape=jax.ShapeDtypeStruct((N_IMG, NSP), jnp.int32),
    )(targets)


@functools.cache
def _make_sc_gather():
    mesh = plsc.VectorSubcoreMesh(core_axis_name="c", subcore_axis_name="s")

    @functools.partial(
        pl.kernel,
        mesh=mesh,
        out_type=jax.ShapeDtypeStruct((N_IMG * HW,), jnp.int32),
        scratch_types=[
            pltpu.VMEM((NSP,), jnp.int32),
            pltpu.VMEM((_CHUNK,), jnp.int32),
            pltpu.VMEM((_CHUNK,), jnp.int32),
            pltpu.VMEM((_CHUNK,), jnp.int32),
        ],
        compiler_params=pltpu.CompilerParams(needs_layout_passes=False),
    )
    def sc_gather(bits_hbm, sp_hbm, spm_hbm, out_hbm, table_v, idx_v, spm_v, out_v):
        wid = lax.axis_index("s") * _NC + lax.axis_index("c")
        img = wid // _W_PER_IMG
        base = pl.multiple_of(wid * _CHUNK, 8)
        tab_off = pl.multiple_of(img * NSP, 8)
        pltpu.sync_copy(bits_hbm.at[pl.ds(tab_off, NSP)], table_v)
        pltpu.sync_copy(sp_hbm.at[pl.ds(base, _CHUNK)], idx_v)
        pltpu.sync_copy(spm_hbm.at[pl.ds(base, _CHUNK)], spm_v)

        def body(i, carry):
            b = i * _LANES
            idx = idx_v[pl.ds(b, _LANES)]
            g = plsc.load_gather(table_v, [idx])
            out_v[pl.ds(b, _LANES)] = g * spm_v[pl.ds(b, _LANES)]
            return carry

        lax.fori_loop(0, _CHUNK // _LANES, body, 0)
        pltpu.sync_copy(out_v, out_hbm.at[pl.ds(base, _CHUNK)])

    return sc_gather


def _math_body(x_ref, m_ref, out_ref, acc_ref, cnt_ref):
    step = pl.program_id(0) * _NBLK + pl.program_id(1)

    @pl.when(step == 0)
    def _():
        acc_ref[0] = jnp.float32(0.0)
        cnt_ref[0] = jnp.int32(0)

    x = x_ref[0]
    m = m_ref[...]
    mb = jnp.broadcast_to(m, (C, _S, 512))
    cbit = lax.broadcasted_iota(jnp.int32, (C, _S, 512), 0)
    t = ((mb >> cbit) & 1).astype(jnp.float32)

    e = jnp.exp(x)
    s0 = jnp.sum(e * (1.0 - t), axis=0, keepdims=True)
    ce = (jnp.log(s0 + e) - x) * t
    pix = jnp.sum(ce, axis=0, keepdims=True)
    k = jnp.sum(t, axis=0, keepdims=True)
    sel = m != 0
    pix_ce = jnp.where(sel, pix / jnp.maximum(k, 1.0), 0.0)

    acc_ref[0] += jnp.sum(pix_ce)
    cnt_ref[0] += jnp.sum(sel.astype(jnp.int32))

    @pl.when(step == N_IMG * _NBLK - 1)
    def _():
        out_ref[0, 0] = acc_ref[0] / (jnp.int32(1) + cnt_ref[0]).astype(jnp.float32)


def _math(x4, m3):
    return pl.pallas_call(
        _math_body,
        grid=(N_IMG, _NBLK),
        in_specs=[
            pl.BlockSpec((1, C, _S, 512), lambda n, b: (n, 0, b, 0)),
            pl.BlockSpec((1, _S, 512), lambda n, b: (n * _NBLK + b, 0, 0)),
        ],
        out_specs=pl.BlockSpec((1, 1), lambda n, b: (0, 0), memory_space=pltpu.SMEM),
        out_shape=jax.ShapeDtypeStruct((1, 1), jnp.float32),
        scratch_shapes=[
            pltpu.SMEM((1,), jnp.float32),
            pltpu.SMEM((1,), jnp.int32),
        ],
    )(x4, m3)


def kernel(inputs, targets, superpixels, spmasks):
    n, c, h, w = inputs.shape
    sp = superpixels.reshape(-1).astype(jnp.int32)
    spm = spmasks.reshape(-1).astype(jnp.int32)

    bits = _pack_targets(targets).reshape(-1)
    masked = _make_sc_gather()(bits, sp, spm)
    m3 = masked.reshape(n * _NBLK, _S, 512)
    out = _math(inputs, m3)
    return out[0, 0]


# speedup vs baseline: 31.0814x; 1.0253x over previous
import functools

import jax
import jax.numpy as jnp
from jax import lax
from jax.experimental import pallas as pl
from jax.experimental.pallas import tpu as pltpu
from jax.experimental.pallas import tpu_sc as plsc

EPS = 1e-08

N_IMG = 4
C = 20
HW = 512 * 512
NSP = 2048

_NC = 2
_NS = 16
_NW = _NC * _NS
_CHUNK = (N_IMG * HW) // _NW
_W_PER_IMG = HW // _CHUNK
_LANES = 16

_S = 64
_B = _S * 512
_NBLK = HW // _B


def _pack_body(t_ref, bits_ref):
    t = t_ref[...].reshape(N_IMG * NSP, C)
    w = jnp.exp2(lax.broadcasted_iota(jnp.int32, (C, 1), 0).astype(jnp.float32))
    tbit = (t != 0.0).astype(jnp.float32)
    bits = jax.lax.dot_general(tbit, w, (((1,), (0,)), ((), ())),
                               preferred_element_type=jnp.float32)
    bits_ref[...] = bits.astype(jnp.int32).reshape(N_IMG, NSP)


def _pack_targets(targets):
    return pl.pallas_call(
        _pack_body,
        out_shape=jax.ShapeDtypeStruct((N_IMG, NSP), jnp.int32),
    )(targets)


@functools.cache
def _make_sc_gather():
    mesh = plsc.VectorSubcoreMesh(core_axis_name="c", subcore_axis_name="s")

    @functools.partial(
        pl.kernel,
        mesh=mesh,
        out_type=jax.ShapeDtypeStruct((N_IMG * HW,), jnp.int32),
        scratch_types=[
            pltpu.VMEM((NSP,), jnp.int32),
            pltpu.VMEM((_CHUNK,), jnp.int32),
            pltpu.VMEM((_CHUNK,), jnp.int32),
        ],
        compiler_params=pltpu.CompilerParams(needs_layout_passes=False),
    )
    def sc_gather(bits_hbm, sp_hbm, out_hbm, table_v, idx_v, out_v):
        wid = lax.axis_index("s") * _NC + lax.axis_index("c")
        img = wid // _W_PER_IMG
        base = pl.multiple_of(wid * _CHUNK, 8)
        tab_off = pl.multiple_of(img * NSP, 8)
        pltpu.sync_copy(bits_hbm.at[pl.ds(tab_off, NSP)], table_v)
        pltpu.sync_copy(sp_hbm.at[pl.ds(base, _CHUNK)], idx_v)

        def body(i, carry):
            b = i * _LANES
            idx = idx_v[pl.ds(b, _LANES)]
            out_v[pl.ds(b, _LANES)] = plsc.load_gather(table_v, [idx])
            return carry

        lax.fori_loop(0, _CHUNK // _LANES, body, 0)
        pltpu.sync_copy(out_v, out_hbm.at[pl.ds(base, _CHUNK)])

    return sc_gather


def _math_body(x_ref, m_ref, spm_ref, out_ref, acc_ref, cnt_ref):
    step = pl.program_id(0) * _NBLK + pl.program_id(1)

    @pl.when(step == 0)
    def _():
        acc_ref[0] = jnp.float32(0.0)
        cnt_ref[0] = jnp.int32(0)

    x = x_ref[0]
    m = m_ref[...]
    mb = jnp.broadcast_to(m, (C, _S, 512))
    cbit = lax.broadcasted_iota(jnp.int32, (C, _S, 512), 0)
    t = ((mb >> cbit) & 1).astype(jnp.float32)

    e = jnp.exp(x)
    s0 = jnp.sum(e * (1.0 - t), axis=0, keepdims=True)
    ce = (jnp.log(s0 + e) - x) * t
    pix = jnp.sum(ce, axis=0, keepdims=True)
    k = jnp.sum(t, axis=0, keepdims=True)
    sel = (m != 0) & (spm_ref[...] != 0)
    pix_ce = jnp.where(sel, pix / jnp.maximum(k, 1.0), 0.0)

    acc_ref[0] += jnp.sum(pix_ce)
    cnt_ref[0] += jnp.sum(sel.astype(jnp.int32))

    @pl.when(step == N_IMG * _NBLK - 1)
    def _():
        out_ref[0, 0] = acc_ref[0] / (jnp.int32(1) + cnt_ref[0]).astype(jnp.float32)


def _math(x4, m3, spm4):
    return pl.pallas_call(
        _math_body,
        grid=(N_IMG, _NBLK),
        in_specs=[
            pl.BlockSpec((1, C, _S, 512), lambda n, b: (n, 0, b, 0)),
            pl.BlockSpec((1, _S, 512), lambda n, b: (n * _NBLK + b, 0, 0)),
            pl.BlockSpec((1, _S, 512), lambda n, b: (n, b, 0)),
        ],
        out_specs=pl.BlockSpec((1, 1), lambda n, b: (0, 0), memory_space=pltpu.SMEM),
        out_shape=jax.ShapeDtypeStruct((1, 1), jnp.float32),
        scratch_shapes=[
            pltpu.SMEM((1,), jnp.float32),
            pltpu.SMEM((1,), jnp.int32),
        ],
    )(x4, m3, spm4)


def kernel(inputs, targets, superpixels, spmasks):
    n, c, h, w = inputs.shape
    sp = superpixels.reshape(-1).astype(jnp.int32)

    bits = _pack_targets(targets).reshape(-1)
    gathered = _make_sc_gather()(bits, sp)
    m3 = gathered.reshape(n * _NBLK, _S, 512)
    out = _math(inputs, m3, spmasks.astype(jnp.int32))
    return out[0, 0]


# speedup vs baseline: 32.7366x; 1.0533x over previous
import functools

import jax
import jax.numpy as jnp
from jax import lax
from jax.experimental import pallas as pl
from jax.experimental.pallas import tpu as pltpu
from jax.experimental.pallas import tpu_sc as plsc

EPS = 1e-08

N_IMG = 4
C = 20
HW = 512 * 512
NSP = 2048

_NC = 2
_NS = 16
_NW = _NC * _NS
_CHUNK = (N_IMG * HW) // _NW
_W_PER_IMG = HW // _CHUNK
_LANES = 16

_S = 64
_B = _S * 512
_NBLK = HW // _B


def _pack_body(t_ref, bits_ref):
    t = t_ref[...].reshape(N_IMG * NSP, C)
    w0 = (1 << lax.broadcasted_iota(jnp.int32, (C, 1), 0)).astype(jnp.float32)
    w = jnp.concatenate([w0, jnp.ones((C, 1), jnp.float32)], axis=1)
    tbit = (t != 0.0).astype(jnp.float32)
    packed = jax.lax.dot_general(tbit, w, (((1,), (0,)), ((), ())),
                                 preferred_element_type=jnp.float32)
    word = (packed[:, 0:1].astype(jnp.int32)
            | (packed[:, 1:2].astype(jnp.int32) << 20))
    bits_ref[...] = word.reshape(N_IMG, NSP)


def _pack_targets(targets):
    return pl.pallas_call(
        _pack_body,
        out_shape=jax.ShapeDtypeStruct((N_IMG, NSP), jnp.int32),
    )(targets)


@functools.cache
def _make_sc_gather():
    mesh = plsc.VectorSubcoreMesh(core_axis_name="c", subcore_axis_name="s")

    @functools.partial(
        pl.kernel,
        mesh=mesh,
        out_type=jax.ShapeDtypeStruct((N_IMG * HW,), jnp.int32),
        scratch_types=[
            pltpu.VMEM((NSP,), jnp.int32),
            pltpu.VMEM((_CHUNK,), jnp.int32),
            pltpu.VMEM((_CHUNK,), jnp.int32),
        ],
        compiler_params=pltpu.CompilerParams(needs_layout_passes=False),
    )
    def sc_gather(bits_hbm, sp_hbm, out_hbm, table_v, idx_v, out_v):
        wid = lax.axis_index("s") * _NC + lax.axis_index("c")
        img = wid // _W_PER_IMG
        base = pl.multiple_of(wid * _CHUNK, 8)
        tab_off = pl.multiple_of(img * NSP, 8)
        pltpu.sync_copy(bits_hbm.at[pl.ds(tab_off, NSP)], table_v)
        pltpu.sync_copy(sp_hbm.at[pl.ds(base, _CHUNK)], idx_v)

        def body(i, carry):
            b = i * _LANES
            idx = idx_v[pl.ds(b, _LANES)]
            out_v[pl.ds(b, _LANES)] = plsc.load_gather(table_v, [idx])
            return carry

        lax.fori_loop(0, _CHUNK // _LANES, body, 0)
        pltpu.sync_copy(out_v, out_hbm.at[pl.ds(base, _CHUNK)])

    return sc_gather


def _math_body(x_ref, m_ref, spm_ref, out_ref, acc_ref, cnt_ref):
    step = pl.program_id(0) * _NBLK + pl.program_id(1)

    @pl.when(step == 0)
    def _():
        acc_ref[0] = jnp.float32(0.0)
        cnt_ref[0] = jnp.int32(0)

    x = x_ref[0]
    m = m_ref[...]
    mb = jnp.broadcast_to(m, (C, _S, 512))
    cbit = lax.broadcasted_iota(jnp.int32, (C, _S, 512), 0)
    tb = ((mb >> cbit) & 1) != 0

    e = jnp.exp(x)
    s0 = jnp.sum(jnp.where(tb, 0.0, e), axis=0, keepdims=True)
    ce = jnp.where(tb, jnp.log(s0 + e) - x, 0.0)
    pix = jnp.sum(ce, axis=0, keepdims=True)
    k = (m >> 20).astype(jnp.float32)
    sel = (m != 0) & (spm_ref[...] != 0)
    pix_ce = jnp.where(sel, pix / jnp.maximum(k, 1.0), 0.0)

    acc_ref[0] += jnp.sum(pix_ce)
    cnt_ref[0] += jnp.sum(sel.astype(jnp.int32))

    @pl.when(step == N_IMG * _NBLK - 1)
    def _():
        out_ref[0, 0] = acc_ref[0] / (jnp.int32(1) + cnt_ref[0]).astype(jnp.float32)


def _math(x4, m3, spm4):
    return pl.pallas_call(
        _math_body,
        grid=(N_IMG, _NBLK),
        in_specs=[
            pl.BlockSpec((1, C, _S, 512), lambda n, b: (n, 0, b, 0)),
            pl.BlockSpec((1, _S, 512), lambda n, b: (n * _NBLK + b, 0, 0)),
            pl.BlockSpec((1, _S, 512), lambda n, b: (n, b, 0)),
        ],
        out_specs=pl.BlockSpec((1, 1), lambda n, b: (0, 0), memory_space=pltpu.SMEM),
        out_shape=jax.ShapeDtypeStruct((1, 1), jnp.float32),
        scratch_shapes=[
            pltpu.SMEM((1,), jnp.float32),
            pltpu.SMEM((1,), jnp.int32),
        ],
    )(x4, m3, spm4)


def kernel(inputs, targets, superpixels, spmasks):
    n, c, h, w = inputs.shape
    sp = superpixels.reshape(-1).astype(jnp.int32)

    bits = _pack_targets(targets).reshape(-1)
    gathered = _make_sc_gather()(bits, sp)
    m3 = gathered.reshape(n * _NBLK, _S, 512)
    out = _math(inputs, m3, spmasks.astype(jnp.int32))
    return out[0, 0]


# speedup vs baseline: 32.8140x; 1.0024x over previous
import functools

import jax
import jax.numpy as jnp
from jax import lax
from jax.experimental import pallas as pl
from jax.experimental.pallas import tpu as pltpu
from jax.experimental.pallas import tpu_sc as plsc

EPS = 1e-08

N_IMG = 4
C = 20
HW = 512 * 512
NSP = 2048

_NC = 2
_NS = 16
_NW = _NC * _NS
_CHUNK = (N_IMG * HW) // _NW
_W_PER_IMG = HW // _CHUNK
_LANES = 16

_S = 64
_B = _S * 512
_NBLK = HW // _B


def _pack_body(t_ref, bits_ref):
    t = t_ref[...].reshape(N_IMG * NSP, C)
    w0 = (1 << lax.broadcasted_iota(jnp.int32, (C, 1), 0)).astype(jnp.float32)
    w1 = jnp.ones((C, 1), jnp.float32)
    tbit = (t != 0.0).astype(jnp.float32)
    dims = (((1,), (0,)), ((), ()))
    bits = jax.lax.dot_general(tbit, w0, dims,
                               preferred_element_type=jnp.float32)
    cnt = jax.lax.dot_general(tbit, w1, dims,
                              preferred_element_type=jnp.float32)
    word = bits.astype(jnp.int32) | (cnt.astype(jnp.int32) << 20)
    bits_ref[...] = word.reshape(N_IMG, NSP)


def _pack_targets(targets):
    return pl.pallas_call(
        _pack_body,
        out_shape=jax.ShapeDtypeStruct((N_IMG, NSP), jnp.int32),
    )(targets)


@functools.cache
def _make_sc_gather():
    mesh = plsc.VectorSubcoreMesh(core_axis_name="c", subcore_axis_name="s")

    @functools.partial(
        pl.kernel,
        mesh=mesh,
        out_type=jax.ShapeDtypeStruct((N_IMG * HW,), jnp.int32),
        scratch_types=[
            pltpu.VMEM((NSP,), jnp.int32),
            pltpu.VMEM((_CHUNK,), jnp.int32),
            pltpu.VMEM((_CHUNK,), jnp.int32),
        ],
        compiler_params=pltpu.CompilerParams(needs_layout_passes=False),
    )
    def sc_gather(bits_hbm, sp_hbm, out_hbm, table_v, idx_v, out_v):
        wid = lax.axis_index("s") * _NC + lax.axis_index("c")
        img = wid // _W_PER_IMG
        base = pl.multiple_of(wid * _CHUNK, 8)
        tab_off = pl.multiple_of(img * NSP, 8)
        pltpu.sync_copy(bits_hbm.at[pl.ds(tab_off, NSP)], table_v)
        pltpu.sync_copy(sp_hbm.at[pl.ds(base, _CHUNK)], idx_v)

        def body(i, carry):
            b = i * _LANES
            idx = idx_v[pl.ds(b, _LANES)]
            out_v[pl.ds(b, _LANES)] = plsc.load_gather(table_v, [idx])
            return carry

        lax.fori_loop(0, _CHUNK // _LANES, body, 0)
        pltpu.sync_copy(out_v, out_hbm.at[pl.ds(base, _CHUNK)])

    return sc_gather


def _math_body(x_ref, m_ref, spm_ref, out_ref, acc_ref, cnt_ref):
    step = pl.program_id(0) * _NBLK + pl.program_id(1)

    @pl.when(step == 0)
    def _():
        acc_ref[0] = jnp.float32(0.0)
        cnt_ref[0] = jnp.int32(0)

    x = x_ref[0]
    m = m_ref[...]
    mb = jnp.broadcast_to(m, (C, _S, 512))
    cbit = lax.broadcasted_iota(jnp.int32, (C, _S, 512), 0)
    tb = ((mb >> cbit) & 1) != 0

    e = jnp.exp(x)
    s0 = jnp.sum(jnp.where(tb, 0.0, e), axis=0, keepdims=True)
    ce = jnp.where(tb, jnp.log(s0 + e) - x, 0.0)
    pix = jnp.sum(ce, axis=0, keepdims=True)
    k = (m >> 20).astype(jnp.float32)
    sel = (m != 0) & (spm_ref[...] != 0)
    pix_ce = jnp.where(sel, pix / jnp.maximum(k, 1.0), 0.0)

    acc_ref[0] += jnp.sum(pix_ce)
    cnt_ref[0] += jnp.sum(sel.astype(jnp.int32))

    @pl.when(step == N_IMG * _NBLK - 1)
    def _():
        out_ref[0, 0] = acc_ref[0] / (jnp.int32(1) + cnt_ref[0]).astype(jnp.float32)


def _math(x4, m3, spm4):
    return pl.pallas_call(
        _math_body,
        grid=(N_IMG, _NBLK),
        in_specs=[
            pl.BlockSpec((1, C, _S, 512), lambda n, b: (n, 0, b, 0)),
            pl.BlockSpec((1, _S, 512), lambda n, b: (n * _NBLK + b, 0, 0)),
            pl.BlockSpec((1, _S, 512), lambda n, b: (n, b, 0)),
        ],
        out_specs=pl.BlockSpec((1, 1), lambda n, b: (0, 0), memory_space=pltpu.SMEM),
        out_shape=jax.ShapeDtypeStruct((1, 1), jnp.float32),
        scratch_shapes=[
            pltpu.SMEM((1,), jnp.float32),
            pltpu.SMEM((1,), jnp.int32),
        ],
    )(x4, m3, spm4)


def kernel(inputs, targets, superpixels, spmasks):
    n, c, h, w = inputs.shape
    sp = superpixels.reshape(-1).astype(jnp.int32)

    bits = _pack_targets(targets).reshape(-1)
    gathered = _make_sc_gather()(bits, sp)
    m3 = gathered.reshape(n * _NBLK, _S, 512)
    out = _math(inputs, m3, spmasks.astype(jnp.int32))
    return out[0, 0]
